# baseline (speedup 1.0000x reference)
import functools

import jax
import jax.numpy as jnp
from jax import lax
from jax.experimental import pallas as pl
from jax.experimental.pallas import tpu as pltpu
from jax.experimental.pallas import tpu_sc as plsc

N_USERS = 5000
N_ITEMS = 5000
N = N_USERS + N_ITEMS
E = 320000
D = 128
LEAKY = 0.5

NC = 2
NS = 16
NW = NC * NS
L = 16

NP = 10240
EPT = 10240
EP = NW * EPT
K = 128
NSUB = EPT // K
NPAIR = NSUB // 2
RPT = N // NS
ZR = 125
NB = K // L

EBYTES = 3 * K * 4
RBYTES = K * D * 4


def _bcast_lane(v16, lane):
    idx = jnp.full((L,), lane, dtype=jnp.int32)
    return v16.at[idx].get(mode="promise_in_bounds")


_sc_mesh = plsc.VectorSubcoreMesh(core_axis_name="c", subcore_axis_name="s")


@functools.partial(
    pl.kernel,
    out_type=jax.ShapeDtypeStruct((NC, NP, D), jnp.float32),
    mesh=_sc_mesh,
    scratch_types=[
        pltpu.VMEM((3, K), jnp.int32),
        pltpu.VMEM((3, K), jnp.int32),
        pltpu.VMEM((K,), jnp.int32),
        pltpu.VMEM((K,), jnp.int32),
        pltpu.VMEM((K, D), jnp.float32),
        pltpu.VMEM((K, D), jnp.float32),
        pltpu.VMEM((ZR, D), jnp.float32),
        pltpu.VMEM_SHARED((N, D), jnp.float32),
        pltpu.SemaphoreType.DMA,
        pltpu.SemaphoreType.DMA,
        pltpu.SemaphoreType.DMA,
    ],
    compiler_params=pltpu.CompilerParams(
        use_tc_tiling_on_sc=False, needs_layout_passes=False),
)
def _spmm_partial(x_hbm, ed_hbm, out_hbm,
                  eda, edb, sidxa, sidxb, rowsa, rowsb, zero_v, acc_sh,
                  seme, semg, sems):
    c = lax.axis_index("c")
    s = lax.axis_index("s")
    wid = s * NC + c

    def zrow(k, _):
        for r in range(D // L):
            zero_v[k, pl.ds(r * L, L)] = jnp.zeros((L,), jnp.float32)
        return 0
    lax.fori_loop(0, ZR, zrow, 0)
    def zacc(q, _):
        pltpu.sync_copy(zero_v, acc_sh.at[pl.ds(s * RPT + q * ZR, ZR)])
        return 0
    lax.fori_loop(0, RPT // ZR, zacc, 0)
    plsc.subcore_barrier()

    def estart(q, ebuf):
        pltpu.async_copy(ed_hbm.at[wid, q], ebuf, seme)

    def edrain():
        pltpu.make_async_copy(ed_hbm.at[wid, 0], eda, seme).wait()

    def gstart(q, ebuf, rows):
        pltpu.async_copy(x_hbm.at[ebuf.at[0]], rows, semg)

    def gdrain():
        pltpu.make_async_copy(x_hbm.at[pl.ds(0, K)], rowsa, semg).wait()

    def sstart(rows, sidx1):
        pltpu.async_copy(rows, acc_sh.at[sidx1], sems, add=True)

    def sdrain():
        pltpu.make_async_copy(rowsa, acc_sh.at[pl.ds(0, K)], sems).wait()

    def scale(ebuf, rows, sidx1):
        for b in range(NB):
            sidx1[pl.ds(b * L, L)] = ebuf[1, pl.ds(b * L, L)]
        def scale16(b, _):
            v16 = plsc.bitcast(ebuf[2, pl.ds(b * L, L)], jnp.float32)
            for l in range(L):
                bc = _bcast_lane(v16, l)
                k = b * L + l
                for r in range(D // L):
                    sl = pl.ds(r * L, L)
                    rows[k, sl] = rows[k, sl] * bc
            return 0
        lax.fori_loop(0, NB, scale16, 0)

    def body(q, _):
        estart(q, eda)
        edrain()
        gstart(q, eda, rowsa)
        gdrain()
        scale(eda, rowsa, sidxa)
        sstart(rowsa, sidxa)
        sdrain()
        return 0
    lax.fori_loop(0, NSUB, body, 0)

    plsc.subcore_barrier()

    for q in range(RPT // ZR):
        off = s * RPT + q * ZR
        pltpu.sync_copy(acc_sh.at[pl.ds(off, ZR)],
                        out_hbm.at[c, pl.ds(off, ZR)])


def _combine(p, leaky):
    def body(p_ref, o_ref):
        x = p_ref[0] + p_ref[1]
        if leaky:
            x = jnp.where(x >= 0, x, LEAKY * x)
        o_ref[...] = x

    rows = 1024
    return pl.pallas_call(
        body,
        out_shape=jax.ShapeDtypeStruct((NP, D), jnp.float32),
        grid=(NP // rows,),
        in_specs=[pl.BlockSpec((2, rows, D), lambda i: (0, i, 0))],
        out_specs=pl.BlockSpec((rows, D), lambda i: (i, 0)),
    )(p)


def kernel(user_emb, item_emb, edge_index, adj_vals):
    x = jnp.concatenate([
        user_emb, item_emb,
        jnp.zeros((NP - N, D), jnp.float32)], axis=0)
    pad = EP - E

    def chunked(a):
        return jnp.concatenate(
            [a, jnp.zeros((pad,), a.dtype)]).reshape(NW, NSUB, K)

    ri = chunked(edge_index[0])
    ci = chunked(edge_index[1])
    vi = chunked(jax.lax.bitcast_convert_type(adj_vals, jnp.int32))
    ed_fwd = jnp.stack([ri, ci, vi], axis=2)
    ed_bwd = jnp.stack([ci, ri, vi], axis=2)

    for _ in range(2):
        p = _spmm_partial(x, ed_fwd)
        t = _combine(p, leaky=False)
        p = _spmm_partial(t, ed_bwd)
        x = _combine(p, leaky=True)

    return x[:N_USERS], x[N_USERS:N]


# speedup vs baseline: 1.0593x; 1.0593x over previous
import functools

import jax
import jax.numpy as jnp
from jax import lax
from jax.experimental import pallas as pl
from jax.experimental.pallas import tpu as pltpu
from jax.experimental.pallas import tpu_sc as plsc

N_USERS = 5000
N_ITEMS = 5000
N = N_USERS + N_ITEMS
E = 320000
D = 128
LEAKY = 0.5

NC = 2
NS = 16
NW = NC * NS
L = 16

NP = 10240
EPT = 10240
EP = NW * EPT
K = 256
KR = K // 128
NSUB = EPT // K
RPT = N // NS
ZR = 125
NB = K // L

EBYTES = 3 * K * 4
RBYTES = K * D * 4


def _bcast_lane(v16, lane):
    idx = jnp.full((L,), lane, dtype=jnp.int32)
    return v16.at[idx].get(mode="promise_in_bounds")


_sc_mesh = plsc.VectorSubcoreMesh(core_axis_name="c", subcore_axis_name="s")


@functools.partial(
    pl.kernel,
    out_type=jax.ShapeDtypeStruct((NC, NP, D), jnp.float32),
    mesh=_sc_mesh,
    scratch_types=[
        pltpu.VMEM((3, K), jnp.int32),
        pltpu.VMEM((3, K), jnp.int32),
        pltpu.VMEM((K,), jnp.int32),
        pltpu.VMEM((K,), jnp.int32),
        pltpu.VMEM((K, D), jnp.float32),
        pltpu.VMEM((ZR, D), jnp.float32),
        pltpu.VMEM_SHARED((N, D), jnp.float32),
        pltpu.SemaphoreType.DMA,
        pltpu.SemaphoreType.DMA,
        pltpu.SemaphoreType.DMA,
    ],
    compiler_params=pltpu.CompilerParams(
        use_tc_tiling_on_sc=False, needs_layout_passes=False),
)
def _spmm_partial(x_hbm, ed_hbm, out_hbm,
                  eda, edb, sidxa, sidxb, rowsa, zero_v, acc_sh,
                  seme, semg, sems):
    c = lax.axis_index("c")
    s = lax.axis_index("s")
    wid = s * NC + c

    def zrow(k, _):
        for r in range(D // L):
            zero_v[k, pl.ds(r * L, L)] = jnp.zeros((L,), jnp.float32)
        return 0
    lax.fori_loop(0, ZR, zrow, 0)
    def zacc(q, _):
        pltpu.sync_copy(zero_v, acc_sh.at[pl.ds(s * RPT + q * ZR, ZR)])
        return 0
    lax.fori_loop(0, RPT // ZR, zacc, 0)
    plsc.subcore_barrier()

    def estart(q, ebuf):
        pltpu.async_copy(ed_hbm.at[wid, q], ebuf, seme)

    def edrain():
        pltpu.make_async_copy(ed_hbm.at[wid, 0], eda, seme).wait()

    def gstart(q, ebuf, rows):
        pltpu.async_copy(x_hbm.at[ebuf.at[0]], rows, semg)

    def gdrain():
        pltpu.make_async_copy(x_hbm.at[pl.ds(0, K)], rowsa, semg).wait()

    def sstart(rows, sidx1):
        pltpu.async_copy(rows, acc_sh.at[sidx1], sems, add=True)

    def sdrain():
        pltpu.make_async_copy(rowsa, acc_sh.at[pl.ds(0, K)], sems).wait()

    def scale(ebuf, rows, sidx1):
        def sidx16(b, _):
            sl = pl.ds(b * L, L)
            sidx1[sl] = ebuf[1, sl]
            return 0
        lax.fori_loop(0, NB, sidx16, 0)
        def scale16(b, _):
            v16 = plsc.bitcast(ebuf[2, pl.ds(b * L, L)], jnp.float32)
            for l in range(L):
                bc = _bcast_lane(v16, l)
                k = b * L + l
                for r in range(D // L):
                    sl = pl.ds(r * L, L)
                    rows[k, sl] = rows[k, sl] * bc
            return 0
        lax.fori_loop(0, NB, scale16, 0)

    def body(q, _):
        estart(q, eda)
        edrain()
        gstart(q, eda, rowsa)
        gdrain()
        scale(eda, rowsa, sidxa)
        sstart(rowsa, sidxa)
        sdrain()
        return 0
    lax.fori_loop(0, NSUB, body, 0)

    plsc.subcore_barrier()

    for q in range(RPT // ZR):
        off = s * RPT + q * ZR
        pltpu.sync_copy(acc_sh.at[pl.ds(off, ZR)],
                        out_hbm.at[c, pl.ds(off, ZR)])


def _combine(p, leaky):
    def body(p_ref, o_ref):
        x = p_ref[0] + p_ref[1]
        if leaky:
            x = jnp.where(x >= 0, x, LEAKY * x)
        o_ref[...] = x

    rows = 1024
    return pl.pallas_call(
        body,
        out_shape=jax.ShapeDtypeStruct((NP, D), jnp.float32),
        grid=(NP // rows,),
        in_specs=[pl.BlockSpec((2, rows, D), lambda i: (0, i, 0))],
        out_specs=pl.BlockSpec((rows, D), lambda i: (i, 0)),
    )(p)


def kernel(user_emb, item_emb, edge_index, adj_vals):
    x = jnp.concatenate([
        user_emb, item_emb,
        jnp.zeros((NP - N, D), jnp.float32)], axis=0)
    pad = EP - E

    def chunked(a):
        return jnp.concatenate(
            [a, jnp.zeros((pad,), a.dtype)]).reshape(NW, NSUB, K)

    ri = chunked(edge_index[0])
    ci = chunked(edge_index[1])
    vi = chunked(jax.lax.bitcast_convert_type(adj_vals, jnp.int32))
    ed_fwd = jnp.stack([ri, ci, vi], axis=2)
    ed_bwd = jnp.stack([ci, ri, vi], axis=2)

    for _ in range(2):
        p = _spmm_partial(x, ed_fwd)
        t = _combine(p, leaky=False)
        p = _spmm_partial(t, ed_bwd)
        x = _combine(p, leaky=True)

    return x[:N_USERS], x[N_USERS:N]


# speedup vs baseline: 1.0607x; 1.0013x over previous
import functools

import jax
import jax.numpy as jnp
from jax import lax
from jax.experimental import pallas as pl
from jax.experimental.pallas import tpu as pltpu
from jax.experimental.pallas import tpu_sc as plsc

N_USERS = 5000
N_ITEMS = 5000
N = N_USERS + N_ITEMS
E = 320000
D = 128
LEAKY = 0.5

NC = 2
NS = 16
NW = NC * NS
L = 16

NP = 10240
EPT = 10240
EP = NW * EPT
K = 256
KR = K // 128
NSUB = EPT // K
RPT = N // NS
ZR = 125
NB = K // L

EBYTES = 3 * K * 4
RBYTES = K * D * 4


def _bcast_lane(v16, lane):
    idx = jnp.full((L,), lane, dtype=jnp.int32)
    return v16.at[idx].get(mode="promise_in_bounds")


_sc_mesh = plsc.VectorSubcoreMesh(core_axis_name="c", subcore_axis_name="s")


@functools.partial(
    pl.kernel,
    out_type=jax.ShapeDtypeStruct((NC, NP, D), jnp.float32),
    mesh=_sc_mesh,
    scratch_types=[
        pltpu.VMEM((3, K), jnp.int32),
        pltpu.VMEM((3, K), jnp.int32),
        pltpu.VMEM((K,), jnp.int32),
        pltpu.VMEM((K,), jnp.int32),
        pltpu.VMEM((K, D), jnp.float32),
        pltpu.VMEM((ZR, D), jnp.float32),
        pltpu.VMEM_SHARED((N, D), jnp.float32),
        pltpu.SemaphoreType.DMA,
        pltpu.SemaphoreType.DMA,
        pltpu.SemaphoreType.DMA,
    ],
    compiler_params=pltpu.CompilerParams(use_tc_tiling_on_sc=False),
)
def _spmm_partial(x_hbm, ed_hbm, out_hbm,
                  eda, edb, sidxa, sidxb, rowsa, zero_v, acc_sh,
                  seme, semg, sems):
    c = lax.axis_index("c")
    s = lax.axis_index("s")
    wid = s * NC + c

    def zrow(k, _):
        for r in range(D // L):
            zero_v[k, pl.ds(r * L, L)] = jnp.zeros((L,), jnp.float32)
        return 0
    lax.fori_loop(0, ZR, zrow, 0)
    def zacc(q, _):
        pltpu.sync_copy(zero_v, acc_sh.at[pl.ds(s * RPT + q * ZR, ZR)])
        return 0
    lax.fori_loop(0, RPT // ZR, zacc, 0)
    plsc.subcore_barrier()

    def estart(q, ebuf):
        pltpu.async_copy(ed_hbm.at[wid, q], ebuf, seme)

    def edrain():
        pltpu.make_async_copy(ed_hbm.at[wid, 0], eda, seme).wait()

    def gstart(q, ebuf, rows):
        pltpu.async_copy(x_hbm.at[ebuf.at[0]], rows, semg)

    def gdrain():
        pltpu.make_async_copy(x_hbm.at[pl.ds(0, K)], rowsa, semg).wait()

    def sstart(rows, sidx1):
        pltpu.async_copy(rows, acc_sh.at[sidx1], sems, add=True)

    def sdrain():
        pltpu.make_async_copy(rowsa, acc_sh.at[pl.ds(0, K)], sems).wait()

    def scale(ebuf, rows, sidx1):
        def sidx16(b, _):
            sl = pl.ds(b * L, L)
            sidx1[sl] = ebuf[1, sl]
            return 0
        lax.fori_loop(0, NB, sidx16, 0)
        def scale16(b, _):
            v16 = lax.bitcast_convert_type(
                ebuf[2, pl.ds(b * L, L)], jnp.float32)
            for l in range(L):
                bc = _bcast_lane(v16, l)
                k = b * L + l
                for r in range(D // L):
                    sl = pl.ds(r * L, L)
                    rows[k, sl] = rows[k, sl] * bc
            return 0
        lax.fori_loop(0, NB, scale16, 0)

    def body(q, _):
        estart(q, eda)
        edrain()
        gstart(q, eda, rowsa)
        gdrain()
        scale(eda, rowsa, sidxa)
        sstart(rowsa, sidxa)
        sdrain()
        return 0
    lax.fori_loop(0, NSUB, body, 0)

    plsc.subcore_barrier()

    for q in range(RPT // ZR):
        off = s * RPT + q * ZR
        pltpu.sync_copy(acc_sh.at[pl.ds(off, ZR)],
                        out_hbm.at[c, pl.ds(off, ZR)])


def _combine(p, leaky):
    def body(p_ref, o_ref):
        x = p_ref[0] + p_ref[1]
        if leaky:
            x = jnp.where(x >= 0, x, LEAKY * x)
        o_ref[...] = x

    rows = 1024
    return pl.pallas_call(
        body,
        out_shape=jax.ShapeDtypeStruct((NP, D), jnp.float32),
        grid=(NP // rows,),
        in_specs=[pl.BlockSpec((2, rows, D), lambda i: (0, i, 0))],
        out_specs=pl.BlockSpec((rows, D), lambda i: (i, 0)),
    )(p)


def kernel(user_emb, item_emb, edge_index, adj_vals):
    x = jnp.concatenate([
        user_emb, item_emb,
        jnp.zeros((NP - N, D), jnp.float32)], axis=0)
    pad = EP - E

    def chunked(a):
        return jnp.concatenate(
            [a, jnp.zeros((pad,), a.dtype)]).reshape(NW, NSUB, K)

    ri = chunked(edge_index[0])
    ci = chunked(edge_index[1])
    vi = chunked(jax.lax.bitcast_convert_type(adj_vals, jnp.int32))
    ed_fwd = jnp.stack([ri, ci, vi], axis=2)
    ed_bwd = jnp.stack([ci, ri, vi], axis=2)

    for _ in range(2):
        p = _spmm_partial(x, ed_fwd)
        t = _combine(p, leaky=False)
        p = _spmm_partial(t, ed_bwd)
        x = _combine(p, leaky=True)

    return x[:N_USERS], x[N_USERS:N]


# speedup vs baseline: 3.3695x; 3.1767x over previous
import functools

import jax
import jax.numpy as jnp
from jax import lax
from jax.experimental import pallas as pl
from jax.experimental.pallas import tpu as pltpu
from jax.experimental.pallas import tpu_sc as plsc

N_USERS = 5000
N_ITEMS = 5000
N = N_USERS + N_ITEMS
E = 320000
D = 128
LEAKY = 0.5

NC = 2
NS = 16
NW = NC * NS
L = 16

NP = 10240
EPT = E // NW
K = 80
NSUB = EPT // K
NPAIR = NSUB // 2
RPT = N // NS
ZR = 25
ECH = 2000
NB = K // L

RBYTES = K * D * 4


def _bcast_lane(v16, lane):
    idx = jnp.full((L,), lane, dtype=jnp.int32)
    return v16.at[idx].get(mode="promise_in_bounds")


_sc_mesh = plsc.VectorSubcoreMesh(core_axis_name="c", subcore_axis_name="s")


@functools.partial(
    pl.kernel,
    out_type=jax.ShapeDtypeStruct((NC, NP, D), jnp.float32),
    mesh=_sc_mesh,
    scratch_types=[
        pltpu.VMEM((EPT,), jnp.int32),
        pltpu.VMEM((EPT,), jnp.int32),
        pltpu.VMEM((EPT,), jnp.float32),
        pltpu.VMEM((K,), jnp.int32),
        pltpu.VMEM((K,), jnp.int32),
        pltpu.VMEM((K, D), jnp.float32),
        pltpu.VMEM((K, D), jnp.float32),
        pltpu.VMEM_SHARED((N, D), jnp.float32),
        pltpu.SemaphoreType.DMA,
        pltpu.SemaphoreType.DMA,
    ],
    compiler_params=pltpu.CompilerParams(use_tc_tiling_on_sc=False),
)
def _spmm_partial(x_hbm, g_hbm, s_hbm, v_hbm, out_hbm,
                  gidx_v, sidx_v, vals_v, sidxa, sidxb, rowsa, rowsb,
                  acc_sh, semg, sems):
    c = lax.axis_index("c")
    s = lax.axis_index("s")
    wid = s * NC + c

    def eload(q, _):
        sl = pl.ds(q * ECH, ECH)
        pltpu.sync_copy(g_hbm.at[wid, sl], gidx_v.at[sl])
        pltpu.sync_copy(s_hbm.at[wid, sl], sidx_v.at[sl])
        pltpu.sync_copy(v_hbm.at[wid, sl], vals_v.at[sl])
        return 0
    lax.fori_loop(0, EPT // ECH, eload, 0)

    def zrow(k, _):
        for r in range(D // L):
            rowsa[k, pl.ds(r * L, L)] = jnp.zeros((L,), jnp.float32)
        return 0
    lax.fori_loop(0, ZR, zrow, 0)
    def zacc(q, _):
        pltpu.sync_copy(rowsa.at[pl.ds(0, ZR)],
                        acc_sh.at[pl.ds(s * RPT + q * ZR, ZR)])
        return 0
    lax.fori_loop(0, RPT // ZR, zacc, 0)
    plsc.subcore_barrier()

    def gstart(q, rows):
        pltpu.async_copy(x_hbm.at[gidx_v.at[pl.ds(q * K, K)]], rows, semg)

    def gdrain():
        pltpu.make_async_copy(x_hbm.at[pl.ds(0, K)], rowsa, semg).wait()

    def sstart(rows, sidx1):
        pltpu.async_copy(rows, acc_sh.at[sidx1], sems, add=True)

    def sdrain():
        pltpu.make_async_copy(rowsa, acc_sh.at[pl.ds(0, K)], sems).wait()

    def scale(q, rows, sidx1):
        e0 = q * K
        for b in range(NB):
            sidx1[pl.ds(b * L, L)] = sidx_v[pl.ds(e0 + b * L, L)]

        def scale16(b, _):
            v16 = vals_v[pl.ds(e0 + b * L, L)]
            for l in range(L):
                bc = _bcast_lane(v16, l)
                k = b * L + l
                for r in range(D // L):
                    sl = pl.ds(r * L, L)
                    rows[k, sl] = rows[k, sl] * bc
            return 0
        lax.fori_loop(0, NB, scale16, 0)

    gstart(0, rowsa)

    def body(i, _):
        qa = 2 * i
        gdrain()

        @pl.when(i >= 1)
        def _():
            sdrain()
        gstart(qa + 1, rowsb)
        scale(qa, rowsa, sidxa)
        sstart(rowsa, sidxa)

        gdrain()
        sdrain()
        gstart(qa + 2, rowsa)
        scale(qa + 1, rowsb, sidxb)
        sstart(rowsb, sidxb)
        return 0
    lax.fori_loop(0, NPAIR, body, 0)

    gdrain()
    sdrain()
    scale(NSUB - 1, rowsa, sidxa)
    sstart(rowsa, sidxa)
    sdrain()

    plsc.subcore_barrier()

    for q in range(RPT // 125):
        off = s * RPT + q * 125
        pltpu.sync_copy(acc_sh.at[pl.ds(off, 125)],
                        out_hbm.at[c, pl.ds(off, 125)])


def _combine(p, leaky):
    def body(p_ref, o_ref):
        x = p_ref[0] + p_ref[1]
        if leaky:
            x = jnp.where(x >= 0, x, LEAKY * x)
        o_ref[...] = x

    rows = 1024
    return pl.pallas_call(
        body,
        out_shape=jax.ShapeDtypeStruct((NP, D), jnp.float32),
        grid=(NP // rows,),
        in_specs=[pl.BlockSpec((2, rows, D), lambda i: (0, i, 0))],
        out_specs=pl.BlockSpec((rows, D), lambda i: (i, 0)),
    )(p)


def kernel(user_emb, item_emb, edge_index, adj_vals):
    x = jnp.concatenate([
        user_emb, item_emb,
        jnp.zeros((NP - N, D), jnp.float32)], axis=0)
    rows = edge_index[0].reshape(NW, EPT)
    cols = edge_index[1].reshape(NW, EPT)
    vals = adj_vals.reshape(NW, EPT)

    for _ in range(2):
        p = _spmm_partial(x, rows, cols, vals)
        t = _combine(p, leaky=False)
        p = _spmm_partial(t, cols, rows, vals)
        x = _combine(p, leaky=True)

    return x[:N_USERS], x[N_USERS:N]


# speedup vs baseline: 3.5446x; 1.0520x over previous
import functools

import jax
import jax.numpy as jnp
from jax import lax
from jax.experimental import pallas as pl
from jax.experimental.pallas import tpu as pltpu
from jax.experimental.pallas import tpu_sc as plsc

N_USERS = 5000
N_ITEMS = 5000
N = N_USERS + N_ITEMS
E = 320000
D = 128
LEAKY = 0.5

NC = 2
NS = 16
NW = NC * NS
L = 16

NP = 10240
EPT = E // NW
K = 80
NSUB = EPT // K
NPAIR = NSUB // 2
RPT = N // NS
ZR = 25
ECH = 2000
NB = K // L

RBYTES = K * D * 4


def _bcast_lane(v16, lane):
    idx = jnp.full((L,), lane, dtype=jnp.int32)
    return v16.at[idx].get(mode="promise_in_bounds")


_sc_mesh = plsc.VectorSubcoreMesh(core_axis_name="c", subcore_axis_name="s")


@functools.partial(
    pl.kernel,
    out_type=jax.ShapeDtypeStruct((NC, NP, D), jnp.float32),
    mesh=_sc_mesh,
    scratch_types=[
        pltpu.VMEM((EPT,), jnp.int32),
        pltpu.VMEM((EPT,), jnp.int32),
        pltpu.VMEM((EPT,), jnp.float32),
        pltpu.VMEM((K,), jnp.int32),
        pltpu.VMEM((K,), jnp.int32),
        pltpu.VMEM((K, D), jnp.float32),
        pltpu.VMEM((K, D), jnp.float32),
        pltpu.VMEM_SHARED((N, D), jnp.float32),
        pltpu.SemaphoreType.DMA,
        pltpu.SemaphoreType.DMA,
    ],
    compiler_params=pltpu.CompilerParams(use_tc_tiling_on_sc=False),
)
def _spmm_partial(x_hbm, g_hbm, s_hbm, v_hbm, out_hbm,
                  gidx_v, sidx_v, vals_v, sidxa, sidxb, rowsa, rowsb,
                  acc_sh, semg, sems):
    c = lax.axis_index("c")
    s = lax.axis_index("s")
    wid = s * NC + c

    def eload(q, _):
        sl = pl.ds(q * ECH, ECH)
        pltpu.async_copy(g_hbm.at[wid, sl], gidx_v.at[sl], semg)
        pltpu.async_copy(s_hbm.at[wid, sl], sidx_v.at[sl], semg)
        pltpu.async_copy(v_hbm.at[wid, sl], vals_v.at[sl], semg)
        return 0
    lax.fori_loop(0, EPT // ECH, eload, 0)

    def zrow(k, _):
        for r in range(D // L):
            rowsa[k, pl.ds(r * L, L)] = jnp.zeros((L,), jnp.float32)
        return 0
    lax.fori_loop(0, ZR, zrow, 0)
    def edrain(q, _):
        sl = pl.ds(0, ECH)
        pltpu.make_async_copy(g_hbm.at[wid, sl], gidx_v.at[sl], semg).wait()
        return 0
    lax.fori_loop(0, 3 * (EPT // ECH), edrain, 0)
    def zacc(q, _):
        pltpu.async_copy(rowsa.at[pl.ds(0, ZR)],
                         acc_sh.at[pl.ds(s * RPT + q * ZR, ZR)], sems)
        return 0
    lax.fori_loop(0, RPT // ZR, zacc, 0)
    def zdrain(q, _):
        pltpu.make_async_copy(rowsa.at[pl.ds(0, ZR)],
                              acc_sh.at[pl.ds(0, ZR)], sems).wait()
        return 0
    lax.fori_loop(0, RPT // ZR, zdrain, 0)
    plsc.subcore_barrier()

    def gstart(q, rows):
        pltpu.async_copy(x_hbm.at[gidx_v.at[pl.ds(q * K, K)]], rows, semg)

    def gdrain():
        pltpu.make_async_copy(x_hbm.at[pl.ds(0, K)], rowsa, semg).wait()

    def sstart(rows, sidx1):
        pltpu.async_copy(rows, acc_sh.at[sidx1], sems, add=True)

    def sdrain():
        pltpu.make_async_copy(rowsa, acc_sh.at[pl.ds(0, K)], sems).wait()

    def scale(q, rows, sidx1):
        e0 = q * K
        for b in range(NB):
            sidx1[pl.ds(b * L, L)] = sidx_v[pl.ds(e0 + b * L, L)]

        def scale16(b, _):
            v16 = vals_v[pl.ds(e0 + b * L, L)]
            for l in range(L):
                bc = _bcast_lane(v16, l)
                k = b * L + l
                for r in range(D // L):
                    sl = pl.ds(r * L, L)
                    rows[k, sl] = rows[k, sl] * bc
            return 0
        lax.fori_loop(0, NB, scale16, 0)

    gstart(0, rowsa)

    def body(i, _):
        qa = 2 * i
        gdrain()

        @pl.when(i >= 1)
        def _():
            sdrain()
        gstart(qa + 1, rowsb)
        scale(qa, rowsa, sidxa)
        sstart(rowsa, sidxa)

        gdrain()
        sdrain()
        gstart(qa + 2, rowsa)
        scale(qa + 1, rowsb, sidxb)
        sstart(rowsb, sidxb)
        return 0
    lax.fori_loop(0, NPAIR, body, 0)

    gdrain()
    sdrain()
    scale(NSUB - 1, rowsa, sidxa)
    sstart(rowsa, sidxa)
    sdrain()

    plsc.subcore_barrier()

    for q in range(RPT // 125):
        off = s * RPT + q * 125
        pltpu.sync_copy(acc_sh.at[pl.ds(off, 125)],
                        out_hbm.at[c, pl.ds(off, 125)])


def _combine(p, leaky):
    def body(p_ref, o_ref):
        x = p_ref[0] + p_ref[1]
        if leaky:
            x = jnp.where(x >= 0, x, LEAKY * x)
        o_ref[...] = x

    rows = 1024
    return pl.pallas_call(
        body,
        out_shape=jax.ShapeDtypeStruct((NP, D), jnp.float32),
        grid=(NP // rows,),
        in_specs=[pl.BlockSpec((2, rows, D), lambda i: (0, i, 0))],
        out_specs=pl.BlockSpec((rows, D), lambda i: (i, 0)),
    )(p)


def kernel(user_emb, item_emb, edge_index, adj_vals):
    x = jnp.concatenate([
        user_emb, item_emb,
        jnp.zeros((NP - N, D), jnp.float32)], axis=0)
    rows = edge_index[0].reshape(NW, EPT)
    cols = edge_index[1].reshape(NW, EPT)
    vals = adj_vals.reshape(NW, EPT)

    for _ in range(2):
        p = _spmm_partial(x, rows, cols, vals)
        t = _combine(p, leaky=False)
        p = _spmm_partial(t, cols, rows, vals)
        x = _combine(p, leaky=True)

    return x[:N_USERS], x[N_USERS:N]
